# baseline (speedup 1.0000x reference)
import jax
import jax.numpy as jnp
from jax import lax
from jax.experimental import pallas as pl
from jax.experimental.pallas import tpu as pltpu

_N = 1000
_NP = 1024
_B = 4
_S = 128
_K = 300


def _graph_kernel(x_ref, w_ref, b_ref, out_ref):
    hs = [jnp.transpose(x_ref[b], (1, 0)) for b in range(_B)]
    h = jnp.concatenate(hs, axis=1)

    hn = []
    for b in range(_B):
        hb = h[:, b * _S:(b + 1) * _S]
        nrm = jnp.sqrt(jnp.sum(hb * hb, axis=1, keepdims=True))
        hn.append(hb / jnp.maximum(nrm, 1e-12))
    hnc = jnp.concatenate(hn, axis=1)

    sim = lax.dot_general(hnc, hnc, (((1,), (1,)), ((), ())),
                          preferred_element_type=jnp.float32,
                          precision=lax.Precision.DEFAULT)
    sim = sim * jnp.float32(1.0 / _B)

    cidx = lax.broadcasted_iota(jnp.int32, (_NP, _NP), 0)
    ridx = lax.broadcasted_iota(jnp.int32, (_NP, _NP), 1)
    valid = (cidx != ridx) & (cidx < _N) & (ridx < _N)
    sim_sel = jnp.where(valid, sim, -jnp.inf)
    ikey = lax.bitcast_convert_type(sim_sel, jnp.int32)
    key = ikey ^ (lax.shift_right_arithmetic(ikey, 31) & jnp.int32(0x7FFFFFFF))

    one = jnp.int32(1)
    zero = jnp.int32(0)
    kk = jnp.int32(_K)

    cnt_pos = jnp.sum(jnp.where(key >= 0, one, zero), axis=0, keepdims=True)
    pos_hi = cnt_pos >= kk
    p0 = jnp.where(pos_hi, jnp.int32(0), jnp.int32(-2147483648))
    c0 = jnp.where(pos_hi, cnt_pos, jnp.full_like(cnt_pos, _NP))

    def t_body(i, pc):
        p, c = pc
        bit = lax.shift_left(one, 30 - i)
        trial = p | bit
        cnt = jnp.sum(jnp.where(key >= trial, one, zero),
                      axis=0, keepdims=True)
        take = cnt >= kk
        return jnp.where(take, trial, p), jnp.where(take, cnt, c)

    T, c_ge = lax.fori_loop(0, 31, t_body, (p0, c0))

    eq = key == T
    c_eq = jnp.sum(jnp.where(eq, one, zero), axis=0, keepdims=True)
    c_gt = c_ge - c_eq
    need = kk - c_gt

    lane_valid = lax.broadcasted_iota(jnp.int32, (1, _NP), 1) < _N
    tie_rows = jnp.sum(jnp.where((c_eq > need) & lane_valid, one, zero))

    def tie_select(_):
        def i_body(i, pI):
            bit = lax.shift_left(one, 9 - i)
            trial = pI + bit
            g = jnp.sum(jnp.where(eq & (cidx < trial), one, zero),
                        axis=0, keepdims=True)
            return jnp.where(g < need, trial, pI)
        return lax.fori_loop(0, 10, i_body, jnp.zeros((1, _NP), jnp.int32))

    Istar = lax.cond(tie_rows > 0, tie_select,
                     lambda _: jnp.full((1, _NP), jnp.int32(_NP - 1)), None)

    mask = (key > T) | (eq & (cidx <= Istar) & (need > 0))
    MT = jnp.where(mask, sim, 0.0)

    deg = jnp.sum(MT, axis=1, keepdims=True)
    dis = jnp.where(deg > 0, lax.rsqrt(deg), 0.0)

    w = w_ref[...]
    ys = []
    for b in range(_B):
        xw = lax.dot_general(h[:, b * _S:(b + 1) * _S], w,
                             (((1,), (0,)), ((), ())),
                             preferred_element_type=jnp.float32,
                             precision=lax.Precision.DEFAULT)
        ys.append(dis * xw)
    y = jnp.concatenate(ys, axis=1)
    z = lax.dot_general(MT, y, (((1,), (0,)), ((), ())),
                        preferred_element_type=jnp.float32,
                        precision=lax.Precision.DEFAULT)
    for b in range(_B):
        res = dis * z[:, b * _S:(b + 1) * _S] + b_ref[...]
        out_ref[b] = jnp.transpose(res, (1, 0))[:, :_N]


def kernel(x, edge_index, weight, bias):
    del edge_index
    xp = jnp.pad(x, ((0, 0), (0, 0), (0, _NP - _N)))
    out = pl.pallas_call(
        _graph_kernel,
        out_shape=jax.ShapeDtypeStruct((_B, _S, _N), jnp.float32),
    )(xp, weight, bias.reshape(1, _S))
    return out


# speedup vs baseline: 1.0913x; 1.0913x over previous
import jax
import jax.numpy as jnp
from jax import lax
from jax.experimental import pallas as pl
from jax.experimental.pallas import tpu as pltpu

_N = 1000
_NP = 1024
_B = 4
_S = 128
_K = 300


def _graph_kernel(h_ref, w_ref, b_ref, out_ref):
    h = h_ref[...]

    hn = []
    for b in range(_B):
        hb = h[:, b * _S:(b + 1) * _S]
        nrm = jnp.sqrt(jnp.sum(hb * hb, axis=1, keepdims=True))
        hn.append(hb / jnp.maximum(nrm, 1e-12))
    hnc = jnp.concatenate(hn, axis=1)

    sim = lax.dot_general(hnc, hnc, (((1,), (1,)), ((), ())),
                          preferred_element_type=jnp.float32,
                          precision=lax.Precision.DEFAULT)
    sim = sim * jnp.float32(1.0 / _B)

    cidx = lax.broadcasted_iota(jnp.int32, (_NP, _NP), 0)
    ridx = lax.broadcasted_iota(jnp.int32, (_NP, _NP), 1)
    valid = (cidx != ridx) & (cidx < _N) & (ridx < _N)
    sim_sel = jnp.where(valid, sim, -jnp.inf)
    ikey = lax.bitcast_convert_type(sim_sel, jnp.int32)
    key = ikey ^ (lax.shift_right_arithmetic(ikey, 31) & jnp.int32(0x7FFFFFFF))

    one = jnp.int32(1)
    zero = jnp.int32(0)
    kk = jnp.int32(_K)

    cnt_pos = jnp.sum(jnp.where(key >= 0, one, zero), axis=0, keepdims=True)
    pos_hi = cnt_pos >= kk
    p0 = jnp.where(pos_hi, jnp.int32(0), jnp.int32(-2147483648))
    c0 = jnp.where(pos_hi, cnt_pos, jnp.full_like(cnt_pos, _NP))

    def t_body(i, pc):
        p, c = pc
        bit = lax.shift_left(one, 30 - i)
        trial = p | bit
        cnt = jnp.sum(jnp.where(key >= trial, one, zero),
                      axis=0, keepdims=True)
        take = cnt >= kk
        return jnp.where(take, trial, p), jnp.where(take, cnt, c)

    T, c_ge = lax.fori_loop(0, 31, t_body, (p0, c0))

    eq = key == T
    c_eq = jnp.sum(jnp.where(eq, one, zero), axis=0, keepdims=True)
    c_gt = c_ge - c_eq
    need = kk - c_gt

    lane_valid = lax.broadcasted_iota(jnp.int32, (1, _NP), 1) < _N
    tie_rows = jnp.sum(jnp.where((c_eq > need) & lane_valid, one, zero))

    def tie_select(_):
        def i_body(i, pI):
            bit = lax.shift_left(one, 9 - i)
            trial = pI + bit
            g = jnp.sum(jnp.where(eq & (cidx < trial), one, zero),
                        axis=0, keepdims=True)
            return jnp.where(g < need, trial, pI)
        return lax.fori_loop(0, 10, i_body, jnp.zeros((1, _NP), jnp.int32))

    Istar = lax.cond(tie_rows > 0, tie_select,
                     lambda _: jnp.full((1, _NP), jnp.int32(_NP - 1)), None)

    mask = (key > T) | (eq & (cidx <= Istar) & (need > 0))
    MT = jnp.where(mask, sim, 0.0)

    deg = jnp.sum(MT, axis=1, keepdims=True)
    dis = jnp.where(deg > 0, lax.rsqrt(deg), 0.0)

    w = w_ref[...]
    ys = []
    for b in range(_B):
        xw = lax.dot_general(h[:, b * _S:(b + 1) * _S], w,
                             (((1,), (0,)), ((), ())),
                             preferred_element_type=jnp.float32,
                             precision=lax.Precision.DEFAULT)
        ys.append(dis * xw)
    y = jnp.concatenate(ys, axis=1)
    z = lax.dot_general(MT, y, (((1,), (0,)), ((), ())),
                        preferred_element_type=jnp.float32,
                        precision=lax.Precision.DEFAULT)
    for b in range(_B):
        out_ref[:, b * _S:(b + 1) * _S] = (
            dis * z[:, b * _S:(b + 1) * _S] + b_ref[...])


def kernel(x, edge_index, weight, bias):
    del edge_index
    h = jnp.transpose(x, (2, 0, 1)).reshape(_N, _B * _S)
    hp = jnp.pad(h, ((0, _NP - _N), (0, 0)))
    out = pl.pallas_call(
        _graph_kernel,
        out_shape=jax.ShapeDtypeStruct((_NP, _B * _S), jnp.float32),
    )(hp, weight, bias.reshape(1, _S))
    return jnp.transpose(out[:_N].reshape(_N, _B, _S), (1, 2, 0))


# speedup vs baseline: 1.1174x; 1.0239x over previous
import jax
import jax.numpy as jnp
from jax import lax
from jax.experimental import pallas as pl
from jax.experimental.pallas import tpu as pltpu

_N = 1000
_NP = 1024
_B = 4
_S = 128
_K = 300


def _graph_kernel(h_ref, w_ref, b_ref, out_ref):
    h = h_ref[...]

    hn = []
    for b in range(_B):
        hb = h[:, b * _S:(b + 1) * _S]
        nrm = jnp.sqrt(jnp.sum(hb * hb, axis=1, keepdims=True))
        hn.append(hb / jnp.maximum(nrm, 1e-12))
    hnc = jnp.concatenate(hn, axis=1)

    sim = lax.dot_general(hnc, hnc, (((1,), (1,)), ((), ())),
                          preferred_element_type=jnp.float32,
                          precision=lax.Precision.DEFAULT)
    sim = sim * jnp.float32(1.0 / _B)

    cidx = lax.broadcasted_iota(jnp.int32, (_NP, _NP), 0)
    rvec = lax.broadcasted_iota(jnp.int32, (1, _NP), 1)
    valid = (cidx != rvec) & (cidx < _N) & (rvec < _N)
    sim_sel = jnp.where(valid, sim, -jnp.inf)
    ikey = lax.bitcast_convert_type(sim_sel, jnp.int32)
    key = ikey ^ (lax.shift_right_arithmetic(ikey, 31) & jnp.int32(0x7FFFFFFF))

    one = jnp.int32(1)
    zero = jnp.int32(0)
    kk = jnp.int32(_K)

    cnt_pos = jnp.sum(jnp.where(key >= 0, one, zero), axis=0, keepdims=True)
    pos_hi = cnt_pos >= kk
    p0 = jnp.where(pos_hi, jnp.int32(0), jnp.int32(-2147483648))
    c0 = jnp.where(pos_hi, cnt_pos, jnp.full_like(cnt_pos, _NP))

    def t_body(i, pc):
        p, c = pc
        bit = lax.shift_left(one, 30 - i)
        trial = p | bit
        cnt = jnp.sum(jnp.where(key >= trial, one, zero),
                      axis=0, keepdims=True)
        take = cnt >= kk
        return jnp.where(take, trial, p), jnp.where(take, cnt, c)

    T, c_ge = lax.fori_loop(0, 31, t_body, (p0, c0))

    c_eq = jnp.sum(jnp.where(key == T, one, zero), axis=0, keepdims=True)
    c_gt = c_ge - c_eq
    need = kk - c_gt

    lane_valid = lax.broadcasted_iota(jnp.int32, (1, _NP), 1) < _N
    tie_rows = jnp.sum(jnp.where((c_eq > need) & lane_valid, one, zero))

    def tie_select(_):
        def i_body(i, pI):
            bit = lax.shift_left(one, 9 - i)
            trial = pI + bit
            g = jnp.sum(jnp.where((key == T) & (cidx < trial), one, zero),
                        axis=0, keepdims=True)
            return jnp.where(g < need, trial, pI)
        return lax.fori_loop(0, 10, i_body, jnp.zeros((1, _NP), jnp.int32))

    Istar = lax.cond(tie_rows > 0, tie_select,
                     lambda _: jnp.full((1, _NP), jnp.int32(_NP - 1)), None)

    mask = (key > T) | ((key == T) & (cidx <= Istar))
    MT = jnp.where(mask, sim, 0.0)

    deg = jnp.sum(MT, axis=1, keepdims=True)
    dis = jnp.where(deg > 0, lax.rsqrt(deg), 0.0)

    w = w_ref[...]
    ys = []
    for b in range(_B):
        xw = lax.dot_general(h[:, b * _S:(b + 1) * _S], w,
                             (((1,), (0,)), ((), ())),
                             preferred_element_type=jnp.float32,
                             precision=lax.Precision.DEFAULT)
        ys.append(dis * xw)
    y = jnp.concatenate(ys, axis=1)
    z = lax.dot_general(MT, y, (((1,), (0,)), ((), ())),
                        preferred_element_type=jnp.float32,
                        precision=lax.Precision.DEFAULT)
    for b in range(_B):
        out_ref[:, b * _S:(b + 1) * _S] = (
            dis * z[:, b * _S:(b + 1) * _S] + b_ref[...])


def kernel(x, edge_index, weight, bias):
    del edge_index
    h = jnp.transpose(x, (2, 0, 1)).reshape(_N, _B * _S)
    hp = jnp.pad(h, ((0, _NP - _N), (0, 0)))
    out = pl.pallas_call(
        _graph_kernel,
        out_shape=jax.ShapeDtypeStruct((_NP, _B * _S), jnp.float32),
    )(hp, weight, bias.reshape(1, _S))
    return jnp.transpose(out[:_N].reshape(_N, _B, _S), (1, 2, 0))


# speedup vs baseline: 1.1380x; 1.0184x over previous
import jax
import jax.numpy as jnp
from jax import lax
from jax.experimental import pallas as pl
from jax.experimental.pallas import tpu as pltpu

_N = 1000
_NP = 1024
_B = 4
_S = 128
_K = 300


def _graph_kernel(h_ref, w_ref, b_ref, out_ref):
    h = h_ref[...]

    hn = []
    for b in range(_B):
        hb = h[:, b * _S:(b + 1) * _S]
        nrm = jnp.sqrt(jnp.sum(hb * hb, axis=1, keepdims=True))
        hn.append(hb / (jnp.float32(2.0) * jnp.maximum(nrm, 1e-12)))
    hnc = jnp.concatenate(hn, axis=1)

    sim = lax.dot_general(hnc, hnc, (((1,), (1,)), ((), ())),
                          preferred_element_type=jnp.float32,
                          precision=lax.Precision.DEFAULT)

    cidx = lax.broadcasted_iota(jnp.int32, (_NP, _NP), 0)
    rvec = lax.broadcasted_iota(jnp.int32, (1, _NP), 1)
    valid = (cidx != rvec) & (cidx < _N) & (rvec < _N)
    sim_sel = jnp.where(valid, sim, jnp.float32(-2.0))
    ikey = lax.bitcast_convert_type(sim_sel, jnp.int32)
    key = ikey ^ (lax.shift_right_arithmetic(ikey, 31) & jnp.int32(0x7FFFFFFF))

    one = jnp.int32(1)
    zero = jnp.int32(0)
    kk = jnp.int32(_K)

    cnt_pos = jnp.sum(jnp.where(key >= 0, one, zero), axis=0, keepdims=True)
    pos_hi = cnt_pos >= kk
    p0 = jnp.where(pos_hi, jnp.int32(0), jnp.int32(-1073741824))
    c0 = jnp.where(pos_hi, cnt_pos, jnp.full_like(cnt_pos, _NP))

    def t_body(i, pc):
        p, c = pc
        bit = lax.shift_left(one, 29 - i)
        trial = p | bit
        cnt = jnp.sum(jnp.where(key >= trial, one, zero),
                      axis=0, keepdims=True)
        take = cnt >= kk
        return jnp.where(take, trial, p), jnp.where(take, cnt, c)

    T, c_ge = lax.fori_loop(0, 30, t_body, (p0, c0))

    c_eq = jnp.sum(jnp.where(key == T, one, zero), axis=0, keepdims=True)
    c_gt = c_ge - c_eq
    need = kk - c_gt

    lane_valid = rvec < _N
    tie_rows = jnp.sum(jnp.where((c_eq > need) & lane_valid, one, zero))

    def tie_select(_):
        def i_body(i, pI):
            bit = lax.shift_left(one, 9 - i)
            trial = pI + bit
            g = jnp.sum(jnp.where((key == T) & (cidx < trial), one, zero),
                        axis=0, keepdims=True)
            return jnp.where(g < need, trial, pI)
        return lax.fori_loop(0, 10, i_body, jnp.zeros((1, _NP), jnp.int32))

    Istar = lax.cond(tie_rows > 0, tie_select,
                     lambda _: jnp.full((1, _NP), jnp.int32(_NP - 1)), None)

    mask = (key > T) | ((key == T) & (cidx <= Istar))
    MT = jnp.where(mask, sim, 0.0)

    deg = jnp.sum(MT, axis=1, keepdims=True)
    dis = jnp.where(deg > 0, lax.rsqrt(deg), 0.0)

    w = w_ref[...]
    ys = []
    for b in range(_B):
        xw = lax.dot_general(h[:, b * _S:(b + 1) * _S], w,
                             (((1,), (0,)), ((), ())),
                             preferred_element_type=jnp.float32,
                             precision=lax.Precision.DEFAULT)
        ys.append(dis * xw)
    y = jnp.concatenate(ys, axis=1)
    z = lax.dot_general(MT, y, (((1,), (0,)), ((), ())),
                        preferred_element_type=jnp.float32,
                        precision=lax.Precision.DEFAULT)
    for b in range(_B):
        out_ref[:, b * _S:(b + 1) * _S] = (
            dis * z[:, b * _S:(b + 1) * _S] + b_ref[...])


def kernel(x, edge_index, weight, bias):
    del edge_index
    h = jnp.transpose(x, (2, 0, 1)).reshape(_N, _B * _S)
    hp = jnp.pad(h, ((0, _NP - _N), (0, 0)))
    out = pl.pallas_call(
        _graph_kernel,
        out_shape=jax.ShapeDtypeStruct((_NP, _B * _S), jnp.float32),
    )(hp, weight, bias.reshape(1, _S))
    return jnp.transpose(out[:_N].reshape(_N, _B, _S), (1, 2, 0))
